# Initial kernel scaffold; baseline (speedup 1.0000x reference)
import jax
import jax.numpy as jnp
from jax.experimental import pallas as pl


def kernel(h, h_edge, edge_weight, edge_index, phis_W1, phis_b1, phis_W2, phis_b2, phio_W1, phio_b1, phio_W2, phio_b2, phip_W1, phip_b1, phip_W2, phip_b2, ln_g, ln_b, grun_Wih, grun_bih, grun_Whh, grun_bhh, grue_Wih, grue_bih, grue_Whh, grue_bhh, lnf_g, lnf_b):
    raise NotImplementedError("write your pallas kernel here")


# speedup vs baseline: 1.8998x; 1.8998x over previous
import jax
import jax.numpy as jnp
from jax.experimental import pallas as pl
from jax.experimental.pallas import tpu as pltpu

_EPS = 1e-5
_HI = jax.lax.Precision.HIGHEST


def _ln_rows(x, g, b):
    mu = jnp.mean(x, axis=1, keepdims=True)
    var = jnp.mean(jnp.square(x - mu), axis=1, keepdims=True)
    return (x - mu) * jax.lax.rsqrt(var + _EPS) * g + b


def _gru_gates(gi, gh, h, D):
    r = jax.nn.sigmoid(gi[:, :D] + gh[:, :D])
    z = jax.nn.sigmoid(gi[:, D:2 * D] + gh[:, D:2 * D])
    n = jnp.tanh(gi[:, 2 * D:] + r * gh[:, 2 * D:])
    return (1.0 - z) * n + z * h


def _bdot(x, w16):
    return jnp.dot(x.astype(jnp.bfloat16), w16,
                   preferred_element_type=jnp.float32)


def _edge_body_factory(first, final, NN, BN, D):
    BE = BN * (NN - 1)

    def body(*refs):
        ew_ref, si_ref, oi_ref, he_ref = refs[:4]
        k = 4
        if not first:
            hsin_ref = refs[k]
            k += 1
        (h_ref, Ws1_ref, Wo1_ref, bs1_ref, bo1_ref, Ws2_ref, Wo2_ref,
         bs2_ref, bo2_ref, lng_ref, lnb_ref, Wp1_ref, bp1_ref, Wp2_ref,
         bp2_ref, Wih_ref, bih_ref, Whh_ref, bhh_ref) = refs[k:k + 19]
        k += 19
        if final:
            lnfg_ref, lnfb_ref = refs[k:k + 2]
            k += 2
            heln_ref, S_ref = refs[k:k + 2]
        else:
            heout_ref, hsout_ref, S_ref = refs[k:k + 3]

        ew = ew_ref[:, :]
        he = he_ref[:, :]
        iota_n = jax.lax.broadcasted_iota(jnp.int32, (BE, NN), 1)
        ohs = jnp.where(si_ref[:, :] == iota_n, 1.0, 0.0)
        oho = jnp.where(oi_ref[:, :] == iota_n, 1.0, 0.0)
        hfull = h_ref[:, :]
        hs_rows = jnp.dot(ohs, hfull, preferred_element_type=jnp.float32,
                          precision=_HI)
        ho_rows = jnp.dot(oho, hfull, preferred_element_type=jnp.float32,
                          precision=_HI)
        xs = jnp.maximum(_bdot(ew * hs_rows, Ws1_ref[:, :]) + bs1_ref[:, :],
                         0.0)
        xo = jnp.maximum(_bdot(ew * ho_rows, Wo1_ref[:, :]) + bo1_ref[:, :],
                         0.0)
        ms = _bdot(xs, Ws2_ref[:, :]) + bs2_ref[:, :]
        mo = _bdot(xo, Wo2_ref[:, :]) + bo2_ref[:, :]
        mp = _ln_rows(ms + mo, lng_ref[:, :], lnb_ref[:, :])

        he16 = he.astype(jnp.bfloat16)
        p1 = jnp.maximum(
            jnp.dot(he16, Wp1_ref[:, :], preferred_element_type=jnp.float32)
            + bp1_ref[:, :], 0.0)
        p2 = _bdot(p1, Wp2_ref[:, :]) + bp2_ref[:, :]
        wij = ew * p2
        rowi = jax.lax.broadcasted_iota(jnp.int32, (BN, BE), 0)
        coli = jax.lax.broadcasted_iota(jnp.int32, (BN, BE), 1)
        lo = rowi * (NN - 1)
        sel = jnp.where((coli >= lo) & (coli < lo + (NN - 1)), 1.0, 0.0)
        S_ref[:, :] = jnp.dot(sel, wij, preferred_element_type=jnp.float32,
                              precision=_HI)

        gi = _bdot(mp, Wih_ref[:, :]) + bih_ref[:, :]
        gh = jnp.dot(he16, Whh_ref[:, :],
                     preferred_element_type=jnp.float32) + bhh_ref[:, :]
        he_new = _gru_gates(gi, gh, he, D)
        hs_new = he_new if first else hsin_ref[:, :] + he_new
        if final:
            heln_ref[:, :] = _ln_rows(hs_new, lnfg_ref[:, :], lnfb_ref[:, :])
        else:
            heout_ref[:, :] = he_new
            hsout_ref[:, :] = hs_new

    return body


def _edge_call(first, final, NN, BN, D, ew2, si2, oi2, he, hsin, weights):
    E = he.shape[0]
    BE = BN * (NN - 1)
    G = E // BE
    row = lambda i: (i, 0)
    full = lambda i: (0, 0)
    in_specs = [
        pl.BlockSpec((BE, 1), row),
        pl.BlockSpec((BE, 1), row),
        pl.BlockSpec((BE, 1), row),
        pl.BlockSpec((BE, D), row),
    ]
    args = [ew2, si2, oi2, he]
    if not first:
        in_specs.append(pl.BlockSpec((BE, D), row))
        args.append(hsin)
    for w in weights:
        in_specs.append(pl.BlockSpec(w.shape, full))
        args.append(w)
    if final:
        out_shape = [
            jax.ShapeDtypeStruct((E, D), jnp.float32),
            jax.ShapeDtypeStruct((NN, D), jnp.float32),
        ]
        out_specs = [
            pl.BlockSpec((BE, D), row),
            pl.BlockSpec((BN, D), row),
        ]
        aliases = {4: 0}
    else:
        out_shape = [
            jax.ShapeDtypeStruct((E, D), jnp.float32),
            jax.ShapeDtypeStruct((E, D), jnp.float32),
            jax.ShapeDtypeStruct((NN, D), jnp.float32),
        ]
        out_specs = [
            pl.BlockSpec((BE, D), row),
            pl.BlockSpec((BE, D), row),
            pl.BlockSpec((BN, D), row),
        ]
        aliases = {} if first else {3: 0, 4: 1}
    return pl.pallas_call(
        _edge_body_factory(first, final, NN, BN, D),
        grid=(G,),
        in_specs=in_specs,
        out_specs=out_specs,
        out_shape=out_shape,
        input_output_aliases=aliases,
        compiler_params=pltpu.CompilerParams(
            dimension_semantics=("arbitrary",)),
    )(*args)


def _node_body_factory(first, final, D):
    def body(*refs):
        S_ref, h_ref = refs[:2]
        k = 2
        if not first:
            hsin_ref = refs[k]
            k += 1
        Wih_ref, bih_ref, Whh_ref, bhh_ref = refs[k:k + 4]
        k += 4
        if final:
            lnfg_ref, lnfb_ref = refs[k:k + 2]
            k += 2
            hout_ref = refs[k]
        else:
            hnew_ref, hsout_ref = refs[k:k + 2]

        h = h_ref[:, :]
        mn = S_ref[:, :]
        gi = _bdot(mn, Wih_ref[:, :]) + bih_ref[:, :]
        gh = _bdot(h, Whh_ref[:, :]) + bhh_ref[:, :]
        h_new = _gru_gates(gi, gh, h, D)
        hs_new = h_new if first else hsin_ref[:, :] + h_new
        if final:
            hout_ref[:, :] = _ln_rows(hs_new, lnfg_ref[:, :], lnfb_ref[:, :])
        else:
            hnew_ref[:, :] = h_new
            hsout_ref[:, :] = hs_new

    return body


def _node_call(first, final, NN, D, S, h, hsin, weights):
    args = [S, h]
    if not first:
        args.append(hsin)
    args.extend(weights)
    if final:
        out_shape = [jax.ShapeDtypeStruct((NN, D), jnp.float32)]
    else:
        out_shape = [
            jax.ShapeDtypeStruct((NN, D), jnp.float32),
            jax.ShapeDtypeStruct((NN, D), jnp.float32),
        ]
    outs = pl.pallas_call(
        _node_body_factory(first, final, D),
        out_shape=out_shape,
    )(*args)
    return outs[0] if final else outs


def kernel(h, h_edge, edge_weight, edge_index,
           phis_W1, phis_b1, phis_W2, phis_b2,
           phio_W1, phio_b1, phio_W2, phio_b2,
           phip_W1, phip_b1, phip_W2, phip_b2,
           ln_g, ln_b,
           grun_Wih, grun_bih, grun_Whh, grun_bhh,
           grue_Wih, grue_bih, grue_Whh, grue_bhh,
           lnf_g, lnf_b):
    NN, D = h.shape
    E = h_edge.shape[0]
    L = phis_W1.shape[0]
    BN = 8

    ew2 = edge_weight.reshape(E, 1)
    si2 = edge_index[0].reshape(E, 1)
    oi2 = edge_index[1].reshape(E, 1)

    bf = jnp.bfloat16
    Ws1T = jnp.swapaxes(phis_W1, 1, 2).astype(bf)
    Wo1T = jnp.swapaxes(phio_W1, 1, 2).astype(bf)
    Ws2T = jnp.swapaxes(phis_W2, 1, 2).astype(bf)
    Wo2T = jnp.swapaxes(phio_W2, 1, 2).astype(bf)
    Wp1T = jnp.swapaxes(phip_W1, 1, 2).astype(bf)
    Wp2T = jnp.swapaxes(phip_W2, 1, 2).astype(bf)
    WeihT = jnp.swapaxes(grue_Wih, 1, 2).astype(bf)
    WehhT = jnp.swapaxes(grue_Whh, 1, 2).astype(bf)
    WnihT = jnp.swapaxes(grun_Wih, 1, 2).astype(bf)
    WnhhT = jnp.swapaxes(grun_Whh, 1, 2).astype(bf)
    lnfg2 = lnf_g[None, :]
    lnfb2 = lnf_b[None, :]

    he = h_edge
    hs_e = None
    hcur = h
    hs_n = None
    heln = None
    hout = None
    for l in range(L):
        first = l == 0
        final = l == L - 1
        ew_weights = [
            hcur, Ws1T[l], Wo1T[l],
            phis_b1[l][None, :], phio_b1[l][None, :],
            Ws2T[l], Wo2T[l],
            phis_b2[l][None, :], phio_b2[l][None, :],
            ln_g[l][None, :], ln_b[l][None, :],
            Wp1T[l], phip_b1[l][None, :], Wp2T[l], phip_b2[l][None, :],
            WeihT[l], grue_bih[l][None, :], WehhT[l], grue_bhh[l][None, :],
        ]
        if final:
            ew_weights += [lnfg2, lnfb2]
        eouts = _edge_call(first, final, NN, BN, D, ew2, si2, oi2,
                           he, hs_e, ew_weights)
        if final:
            heln, S = eouts
        else:
            he, hs_e, S = eouts
        n_weights = [
            WnihT[l], grun_bih[l][None, :], WnhhT[l], grun_bhh[l][None, :],
        ]
        if final:
            n_weights += [lnfg2, lnfb2]
        nouts = _node_call(first, final, NN, D, S, hcur, hs_n, n_weights)
        if final:
            hout = nouts
        else:
            hcur, hs_n = nouts
    return (hout, heln)
